# baseline (speedup 1.0000x reference)
import functools

import jax
import jax.numpy as jnp
from jax.experimental import pallas as pl
from jax.experimental.pallas import tpu as pltpu
from jax.experimental.pallas import tpu_sc as plsc

_N = 10000
_E = 320000
_CF = 128
_CL = 64
_NUM_GCN = 2
_NUM_LABEL = 10

_NC = 2
_NS = 16
_NW = _NC * _NS

_NP = 10240
_EP = 327680
_K = 128
_EPT = _EP // _NW
_NCH = _EPT // _K
_RPT = _NP // _NS
_RCH = _RPT // _K


def _make_spmm(C):
  kr = 32 if C == 128 else 128
  nch = _EPT // kr
  rch = _RPT // kr
  mesh = plsc.VectorSubcoreMesh(
      core_axis_name="c", subcore_axis_name="s",
      num_cores=_NC, num_subcores=_NS)

  @functools.partial(
      pl.kernel,
      out_type=jax.ShapeDtypeStruct((_NC, _NP, C), jnp.float32),
      mesh=mesh,
      scratch_types=[
          pltpu.VMEM((nch, kr), jnp.int32),
          pltpu.VMEM((nch, kr), jnp.int32),
          pltpu.VMEM((nch, kr), jnp.float32),
          pltpu.VMEM((kr, C), jnp.float32),
          pltpu.VMEM((kr, C), jnp.float32),
          pltpu.VMEM((kr, C), jnp.float32),
          pltpu.VMEM_SHARED((_NP, C), jnp.float32),
          pltpu.SemaphoreType.DMA,
          pltpu.SemaphoreType.DMA,
          pltpu.SemaphoreType.DMA,
          pltpu.SemaphoreType.DMA,
          pltpu.SemaphoreType.DMA,
          pltpu.SemaphoreType.DMA,
      ],
      compiler_params=pltpu.CompilerParams(use_tc_tiling_on_sc=False),
  )
  def spmm(row_h, col_h, w_h, xwp_h, out_h, idx_r, idx_c, wbuf, rows0,
           rows1, rows2, acc, semg0, semg1, semg2, sems0, sems1, sems2):
    cid = jax.lax.axis_index("c")
    sid = jax.lax.axis_index("s")
    tile = cid * _NS + sid
    rows = (rows0, rows1, rows2)
    semg = (semg0, semg1, semg2)
    sems = (sems0, sems1, sems2)

    pltpu.sync_copy(row_h.at[tile], idx_r)
    pltpu.sync_copy(col_h.at[tile], idx_c)
    pltpu.sync_copy(w_h.at[tile], wbuf)

    zero = jnp.zeros((16,), jnp.float32)

    def zrow(i, carry):
      for j in range(C // 16):
        rows0[i, pl.ds(j * 16, 16)] = zero
      return carry

    jax.lax.fori_loop(0, kr, zrow, 0)
    for b in range(rch):
      pltpu.sync_copy(rows0, acc.at[pl.ds(sid * _RPT + b * kr, kr)])
    plsc.subcore_barrier()

    def gather(m, b):
      pltpu.async_copy(xwp_h.at[idx_r.at[m]], rows[b], semg[b])

    def wait_gather(m, b):
      pltpu.make_async_copy(xwp_h.at[idx_r.at[m]], rows[b], semg[b]).wait()

    def scatter(m, b):
      pltpu.sync_copy(rows[b], acc.at[idx_c.at[m]], add=True)

    def scale(m, b):
      rb = rows[b]

      def sgroup(g, c2):
        wv = wbuf[m, pl.ds(g * 16, 16)]
        base = g * 16
        for lane in range(16):
          ws = wv[lane]
          for j in range(C // 16):
            sl = pl.ds(j * 16, 16)
            rb[base + lane, sl] = rb[base + lane, sl] * ws
        return c2

      jax.lax.fori_loop(0, kr // 16, sgroup, 0)

    def step(m, b):
      wait_gather(m, b)
      scale(m, b)
      scatter(m, b)
      bn = (b + 2) % 3

      @pl.when(m + 2 < nch)
      def _():
        gather(m + 2, bn)

    gather(0, 0)
    gather(1, 1)

    def triple(t, carry):
      m0 = t * 3
      step(m0, 0)
      step(m0 + 1, 1)
      step(m0 + 2, 2)
      return carry

    nt = nch // 3
    jax.lax.fori_loop(0, nt, triple, 0)
    for m in range(nt * 3, nch):
      step(jnp.int32(m), m % 3)
    plsc.subcore_barrier()

    for b in range(rch):
      r0 = sid * _RPT + b * kr
      pltpu.sync_copy(acc.at[pl.ds(r0, kr)], rows0)
      pltpu.sync_copy(rows0, out_h.at[cid, pl.ds(r0, kr)])

  return spmm


_spmm_f = _make_spmm(_CF)
_spmm_l = _make_spmm(_CL)


def _tc_matmul(h, w):
  np_, cin = h.shape
  cout = w.shape[1]
  bm = 1024

  def body(h_ref, w_ref, o_ref):
    o_ref[...] = jnp.dot(h_ref[...], w_ref[...],
                         preferred_element_type=jnp.float32)

  return pl.pallas_call(
      body,
      grid=(np_ // bm,),
      in_specs=[
          pl.BlockSpec((bm, cin), lambda i: (i, 0)),
          pl.BlockSpec((cin, cout), lambda i: (0, 0)),
      ],
      out_specs=pl.BlockSpec((bm, cout), lambda i: (i, 0)),
      out_shape=jax.ShapeDtypeStruct((np_, cout), jnp.float32),
  )(h, w)


def _tc_fusion(fp, lp, wf, wl):
  np_ = fp.shape[0]
  cout = wf.shape[1]
  bm = 1024

  def body(fp_ref, lp_ref, wf_ref, wl_ref, o_ref):
    o_ref[...] = (
        jnp.dot(fp_ref[...], wf_ref[...], preferred_element_type=jnp.float32)
        + jnp.dot(lp_ref[...], wl_ref[...],
                  preferred_element_type=jnp.float32))

  return pl.pallas_call(
      body,
      grid=(np_ // bm,),
      in_specs=[
          pl.BlockSpec((bm, fp.shape[1]), lambda i: (i, 0)),
          pl.BlockSpec((bm, lp.shape[1]), lambda i: (i, 0)),
          pl.BlockSpec(wf.shape, lambda i: (0, 0)),
          pl.BlockSpec(wl.shape, lambda i: (0, 0)),
      ],
      out_specs=pl.BlockSpec((bm, cout), lambda i: (i, 0)),
      out_shape=jax.ShapeDtypeStruct((np_, cout), jnp.float32),
  )(fp, lp, wf, wl)


def _layer(h, dinv_p, row_p, col_p, w_p, init_w, pool_p, wih, whh, bih, bhh,
           spmm, c):
  score = (h @ pool_p) / jnp.linalg.norm(pool_p)
  score = jnp.where(jnp.arange(_NP) < _N, score, -jnp.inf)
  vals, idx = jax.lax.top_k(score, c)
  x_t = h[idx] * jnp.tanh(vals)[:, None]
  gx = x_t @ wih.T + bih
  gh = init_w @ whh.T + bhh
  r = jax.nn.sigmoid(gx[:, :c] + gh[:, :c])
  z = jax.nn.sigmoid(gx[:, c:2 * c] + gh[:, c:2 * c])
  ng = jnp.tanh(gx[:, 2 * c:] + r * gh[:, 2 * c:])
  w_ev = (1.0 - z) * ng + z * init_w
  xwp = dinv_p[:, None] * _tc_matmul(h, w_ev)
  parts = spmm(row_p, col_p, w_p, xwp)
  return dinv_p[:, None] * (parts[0] + parts[1] + xwp)


def kernel(x, y, edge_index, edge_weight, f_init_w, f_pool, f_Wih, f_Whh,
           f_bih, f_bhh, l_init_w, l_pool, l_Wih, l_Whh, l_bih, l_bhh,
           fusion_W, fusion_b):
  row = edge_index[0]
  col = edge_index[1]
  deg = jax.ops.segment_sum(edge_weight, col, num_segments=_N) + 1.0
  dinv = jax.lax.rsqrt(deg)
  dinv_p = jnp.pad(dinv, (0, _NP - _N))
  row_p = jnp.pad(row, (0, _EP - _E)).reshape(_NW, _EPT)
  col_p = jnp.pad(col, (0, _EP - _E)).reshape(_NW, _EPT)
  w_p = jnp.pad(edge_weight, (0, _EP - _E)).reshape(_NW, _EPT)

  def _tables(c):
    kr = 32 if c == 128 else 128
    shp = (_NW, _EPT // kr, kr)
    return (row_p.reshape(shp), col_p.reshape(shp), w_p.reshape(shp))

  ef = _tables(_CF)
  el = _tables(_CL)

  h = jnp.pad(x, ((0, _NP - _N), (0, 0)))
  for i in range(_NUM_GCN):
    h = _layer(h, dinv_p, ef[0], ef[1], ef[2], f_init_w[i], f_pool[i],
               f_Wih[i], f_Whh[i], f_bih[i], f_bhh[i], _spmm_f, _CF)
  hl = jnp.pad(y, ((0, _NP - _N), (0, 0)))
  for i in range(_NUM_LABEL):
    hl = _layer(hl, dinv_p, el[0], el[1], el[2], l_init_w[i], l_pool[i],
                l_Wih[i], l_Whh[i], l_bih[i], l_bhh[i], _spmm_l, _CL)

  wf = fusion_W[:, :_CF].T
  wl = fusion_W[:, _CF:].T
  out = _tc_fusion(h, hl, wf, wl)[:_N] + fusion_b
  return out


# speedup vs baseline: 1.0005x; 1.0005x over previous
import functools

import jax
import jax.numpy as jnp
from jax.experimental import pallas as pl
from jax.experimental.pallas import tpu as pltpu
from jax.experimental.pallas import tpu_sc as plsc

_N = 10000
_E = 320000
_CF = 128
_CL = 64
_NUM_GCN = 2
_NUM_LABEL = 10

_NC = 2
_NS = 16
_NW = _NC * _NS

_NP = 10240
_EP = 327680
_K = 128
_EPT = _EP // _NW
_NCH = _EPT // _K
_RPT = _NP // _NS
_RCH = _RPT // _K


def _make_spmm(C):
  kr = 32 if C == 128 else 128
  nch = _EPT // kr
  rch = _RPT // kr
  mesh = plsc.VectorSubcoreMesh(
      core_axis_name="c", subcore_axis_name="s",
      num_cores=_NC, num_subcores=_NS)

  @functools.partial(
      pl.kernel,
      out_type=jax.ShapeDtypeStruct((_NC, _NP, C), jnp.float32),
      mesh=mesh,
      scratch_types=[
          pltpu.VMEM((nch, kr), jnp.int32),
          pltpu.VMEM((nch, kr), jnp.int32),
          pltpu.VMEM((nch, kr), jnp.float32),
          pltpu.VMEM((kr, C), jnp.float32),
          pltpu.VMEM((kr, C), jnp.float32),
          pltpu.VMEM((kr, C), jnp.float32),
          pltpu.VMEM_SHARED((_NP, C), jnp.float32),
          pltpu.SemaphoreType.DMA,
          pltpu.SemaphoreType.DMA,
          pltpu.SemaphoreType.DMA,
          pltpu.SemaphoreType.DMA,
          pltpu.SemaphoreType.DMA,
          pltpu.SemaphoreType.DMA,
      ],
      compiler_params=pltpu.CompilerParams(use_tc_tiling_on_sc=False),
  )
  def spmm(row_h, col_h, w_h, xwp_h, out_h, idx_r, idx_c, wbuf, rows0,
           rows1, rows2, acc, semg0, semg1, semg2, sems0, sems1, sems2):
    cid = jax.lax.axis_index("c")
    sid = jax.lax.axis_index("s")
    tile = cid * _NS + sid
    rows = (rows0, rows1, rows2)
    semg = (semg0, semg1, semg2)
    sems = (sems0, sems1, sems2)

    pltpu.sync_copy(row_h.at[tile], idx_r)
    pltpu.sync_copy(col_h.at[tile], idx_c)
    pltpu.sync_copy(w_h.at[tile], wbuf)

    zero = jnp.zeros((16,), jnp.float32)

    def zrow(i, carry):
      for j in range(C // 16):
        rows0[i, pl.ds(j * 16, 16)] = zero
      return carry

    jax.lax.fori_loop(0, kr, zrow, 0)
    for b in range(rch):
      pltpu.sync_copy(rows0, acc.at[pl.ds(sid * _RPT + b * kr, kr)])
    plsc.subcore_barrier()

    def gather(m, b):
      pltpu.async_copy(xwp_h.at[idx_r.at[m]], rows[b], semg[b])

    def wait_gather(m, b):
      pltpu.make_async_copy(xwp_h.at[idx_r.at[m]], rows[b], semg[b]).wait()

    def scatter(m, b):
      pltpu.sync_copy(rows[b], acc.at[idx_c.at[m]], add=True)

    def scale(m, b):
      rb = rows[b]

      def sgroup(g, c2):
        wv = wbuf[m, pl.ds(g * 16, 16)]
        base = g * 16
        for lane in range(16):
          ws = wv[lane]
          for j in range(C // 16):
            sl = pl.ds(j * 16, 16)
            rb[base + lane, sl] = rb[base + lane, sl] * ws
        return c2

      jax.lax.fori_loop(0, kr // 16, sgroup, 0)

    def step(m, b):
      wait_gather(m, b)
      scale(m, b)
      scatter(m, b)
      bn = (b + 2) % 3

      @pl.when(m + 2 < nch)
      def _():
        gather(m + 2, bn)

    gather(0, 0)
    gather(1, 1)

    def triple(t, carry):
      m0 = t * 3
      step(m0, 0)
      step(m0 + 1, 1)
      step(m0 + 2, 2)
      return carry

    nt = nch // 3
    jax.lax.fori_loop(0, nt, triple, 0)
    for m in range(nt * 3, nch):
      step(jnp.int32(m), m % 3)
    plsc.subcore_barrier()

    for b in range(rch):
      r0 = sid * _RPT + b * kr
      pltpu.sync_copy(acc.at[pl.ds(r0, kr)], rows0)
      pltpu.sync_copy(rows0, out_h.at[cid, pl.ds(r0, kr)])

  return spmm


_spmm_f = _make_spmm(_CF)
_spmm_l = _make_spmm(_CL)


def _tc_matmul(h, w):
  np_, cin = h.shape
  cout = w.shape[1]
  bm = 1024

  def body(h_ref, w_ref, o_ref):
    o_ref[...] = jnp.dot(h_ref[...], w_ref[...],
                         preferred_element_type=jnp.float32)

  return pl.pallas_call(
      body,
      grid=(np_ // bm,),
      in_specs=[
          pl.BlockSpec((bm, cin), lambda i: (i, 0)),
          pl.BlockSpec((cin, cout), lambda i: (0, 0)),
      ],
      out_specs=pl.BlockSpec((bm, cout), lambda i: (i, 0)),
      out_shape=jax.ShapeDtypeStruct((np_, cout), jnp.float32),
  )(h, w)


def _tc_fusion(fp, lp, wf, wl):
  np_ = fp.shape[0]
  cout = wf.shape[1]
  bm = 1024

  def body(fp_ref, lp_ref, wf_ref, wl_ref, o_ref):
    o_ref[...] = (
        jnp.dot(fp_ref[...], wf_ref[...], preferred_element_type=jnp.float32)
        + jnp.dot(lp_ref[...], wl_ref[...],
                  preferred_element_type=jnp.float32))

  return pl.pallas_call(
      body,
      grid=(np_ // bm,),
      in_specs=[
          pl.BlockSpec((bm, fp.shape[1]), lambda i: (i, 0)),
          pl.BlockSpec((bm, lp.shape[1]), lambda i: (i, 0)),
          pl.BlockSpec(wf.shape, lambda i: (0, 0)),
          pl.BlockSpec(wl.shape, lambda i: (0, 0)),
      ],
      out_specs=pl.BlockSpec((bm, cout), lambda i: (i, 0)),
      out_shape=jax.ShapeDtypeStruct((np_, cout), jnp.float32),
  )(fp, lp, wf, wl)


def _layer(h, dinv_p, row_p, col_p, w_p, init_w, pool_p, wih, whh, bih, bhh,
           spmm, c):
  score = (h @ pool_p) / jnp.linalg.norm(pool_p)
  score = jnp.where(jnp.arange(_NP) < _N, score, -jnp.inf)
  vals, idx = jax.lax.top_k(score, c)
  x_t = h[idx] * jnp.tanh(vals)[:, None]
  gx = x_t @ wih.T + bih
  gh = init_w @ whh.T + bhh
  r = jax.nn.sigmoid(gx[:, :c] + gh[:, :c])
  z = jax.nn.sigmoid(gx[:, c:2 * c] + gh[:, c:2 * c])
  ng = jnp.tanh(gx[:, 2 * c:] + r * gh[:, 2 * c:])
  w_ev = (1.0 - z) * ng + z * init_w
  xwp = dinv_p[:, None] * _tc_matmul(h, w_ev)
  parts = spmm(row_p, col_p, w_p, xwp)
  return dinv_p[:, None] * (parts[0] + parts[1] + xwp)


def kernel(x, y, edge_index, edge_weight, f_init_w, f_pool, f_Wih, f_Whh,
           f_bih, f_bhh, l_init_w, l_pool, l_Wih, l_Whh, l_bih, l_bhh,
           fusion_W, fusion_b):
  row = edge_index[0]
  col = edge_index[1]
  deg = jax.ops.segment_sum(edge_weight, col, num_segments=_N) + 1.0
  dinv = jax.lax.rsqrt(deg)
  dinv_p = jnp.pad(dinv, (0, _NP - _N))
  row_p = jnp.pad(row, (0, _EP - _E)).reshape(_NW, _EPT)
  col_p = jnp.pad(col, (0, _EP - _E)).reshape(_NW, _EPT)
  w_p = jnp.pad(edge_weight, (0, _EP - _E)).reshape(_NW, _EPT)

  def _tables(c):
    kr = 32 if c == 128 else 128
    shp = (_NW, _EPT // kr, kr)
    return (row_p.reshape(shp), col_p.reshape(shp), w_p.reshape(shp))

  ef = _tables(_CF)
  el = _tables(_CL)

  h = jnp.pad(x, ((0, _NP - _N), (0, 0)))
  hl = jnp.pad(y, ((0, _NP - _N), (0, 0)))
  fi = 0
  for i in range(_NUM_LABEL):
    if fi < _NUM_GCN:
      h = _layer(h, dinv_p, ef[0], ef[1], ef[2], f_init_w[fi], f_pool[fi],
                 f_Wih[fi], f_Whh[fi], f_bih[fi], f_bhh[fi], _spmm_f, _CF)
      fi += 1
    hl = _layer(hl, dinv_p, el[0], el[1], el[2], l_init_w[i], l_pool[i],
                l_Wih[i], l_Whh[i], l_bih[i], l_bhh[i], _spmm_l, _CL)

  wf = fusion_W[:, :_CF].T
  wl = fusion_W[:, _CF:].T
  out = _tc_fusion(h, hl, wf, wl)[:_N] + fusion_b
  return out


# speedup vs baseline: 1.0238x; 1.0233x over previous
import functools

import jax
import jax.numpy as jnp
from jax.experimental import pallas as pl
from jax.experimental.pallas import tpu as pltpu
from jax.experimental.pallas import tpu_sc as plsc

_N = 10000
_E = 320000
_CF = 128
_CL = 64
_NUM_GCN = 2
_NUM_LABEL = 10

_NC = 2
_NS = 16
_NW = _NC * _NS

_NP = 10240
_EP = 327680
_K = 128
_EPT = _EP // _NW
_NCH = _EPT // _K
_RPT = _NP // _NS
_RCH = _RPT // _K


def _make_spmm(C):
  kr = 32 if C == 128 else 128
  nch = _EPT // kr
  rch = _RPT // kr
  mesh = plsc.VectorSubcoreMesh(
      core_axis_name="c", subcore_axis_name="s",
      num_cores=_NC, num_subcores=_NS)

  @functools.partial(
      pl.kernel,
      out_type=jax.ShapeDtypeStruct((_NC, _NP, C), jnp.float32),
      mesh=mesh,
      scratch_types=[
          pltpu.VMEM((nch, kr), jnp.int32),
          pltpu.VMEM((nch, kr), jnp.int32),
          pltpu.VMEM((nch, kr), jnp.float32),
          pltpu.VMEM((kr, C), jnp.float32),
          pltpu.VMEM((kr, C), jnp.float32),
          pltpu.VMEM((kr, C), jnp.float32),
          pltpu.VMEM_SHARED((_NP, C), jnp.float32),
          pltpu.SemaphoreType.DMA,
          pltpu.SemaphoreType.DMA,
          pltpu.SemaphoreType.DMA,
          pltpu.SemaphoreType.DMA,
          pltpu.SemaphoreType.DMA,
          pltpu.SemaphoreType.DMA,
      ],
      compiler_params=pltpu.CompilerParams(use_tc_tiling_on_sc=False),
  )
  def spmm(row_h, col_h, w_h, xwp_h, out_h, idx_r, idx_c, wbuf, rows0,
           rows1, rows2, acc, semg0, semg1, semg2, sems0, sems1, sems2):
    cid = jax.lax.axis_index("c")
    sid = jax.lax.axis_index("s")
    tile = cid * _NS + sid
    rows = (rows0, rows1, rows2)
    semg = (semg0, semg1, semg2)
    sems = (sems0, sems1, sems2)

    pltpu.sync_copy(row_h.at[tile], idx_r)
    pltpu.sync_copy(col_h.at[tile], idx_c)
    pltpu.sync_copy(w_h.at[tile], wbuf)

    zero = jnp.zeros((16,), jnp.float32)

    def zrow(i, carry):
      for j in range(C // 16):
        rows0[i, pl.ds(j * 16, 16)] = zero
      return carry

    jax.lax.fori_loop(0, kr, zrow, 0)
    for b in range(rch):
      pltpu.sync_copy(rows0, acc.at[pl.ds(sid * _RPT + b * kr, kr)])
    plsc.subcore_barrier()

    def gather(m, b):
      pltpu.async_copy(xwp_h.at[idx_r.at[m]], rows[b], semg[b])

    def wait_gather(m, b):
      pltpu.make_async_copy(xwp_h.at[idx_r.at[m]], rows[b], semg[b]).wait()

    def scatter(m, b):
      pltpu.async_copy(rows[b], acc.at[idx_c.at[m]], sems[b], add=True)

    def wait_scatter(m, b):
      pltpu.make_async_copy(rows[b], acc.at[idx_c.at[m]], sems[b]).wait()

    def scale(m, b):
      rb = rows[b]

      def sgroup(g, c2):
        wv = wbuf[m, pl.ds(g * 16, 16)]
        base = g * 16
        for lane in range(16):
          ws = wv[lane]
          for j in range(C // 16):
            sl = pl.ds(j * 16, 16)
            rb[base + lane, sl] = rb[base + lane, sl] * ws
        return c2

      jax.lax.fori_loop(0, kr // 16, sgroup, 0)

    def step(m, b):
      wait_gather(m, b)
      scale(m, b)
      bn = (b + 2) % 3

      @pl.when(m > 0)
      def _():
        wait_scatter(m - 1, bn)

      scatter(m, b)

      @pl.when(m + 2 < nch)
      def _():
        gather(m + 2, bn)

    gather(0, 0)
    gather(1, 1)

    def triple(t, carry):
      m0 = t * 3
      step(m0, 0)
      step(m0 + 1, 1)
      step(m0 + 2, 2)
      return carry

    nt = nch // 3
    jax.lax.fori_loop(0, nt, triple, 0)
    for m in range(nt * 3, nch):
      step(jnp.int32(m), m % 3)
    wait_scatter(jnp.int32(nch - 1), (nch - 1) % 3)
    plsc.subcore_barrier()

    for b in range(rch):
      r0 = sid * _RPT + b * kr
      pltpu.sync_copy(acc.at[pl.ds(r0, kr)], rows0)
      pltpu.sync_copy(rows0, out_h.at[cid, pl.ds(r0, kr)])

  return spmm


_spmm_f = _make_spmm(_CF)
_spmm_l = _make_spmm(_CL)


def _tc_matmul(h, w):
  np_, cin = h.shape
  cout = w.shape[1]
  bm = 1024

  def body(h_ref, w_ref, o_ref):
    o_ref[...] = jnp.dot(h_ref[...], w_ref[...],
                         preferred_element_type=jnp.float32)

  return pl.pallas_call(
      body,
      grid=(np_ // bm,),
      in_specs=[
          pl.BlockSpec((bm, cin), lambda i: (i, 0)),
          pl.BlockSpec((cin, cout), lambda i: (0, 0)),
      ],
      out_specs=pl.BlockSpec((bm, cout), lambda i: (i, 0)),
      out_shape=jax.ShapeDtypeStruct((np_, cout), jnp.float32),
  )(h, w)


def _tc_fusion(fp, lp, wf, wl):
  np_ = fp.shape[0]
  cout = wf.shape[1]
  bm = 1024

  def body(fp_ref, lp_ref, wf_ref, wl_ref, o_ref):
    o_ref[...] = (
        jnp.dot(fp_ref[...], wf_ref[...], preferred_element_type=jnp.float32)
        + jnp.dot(lp_ref[...], wl_ref[...],
                  preferred_element_type=jnp.float32))

  return pl.pallas_call(
      body,
      grid=(np_ // bm,),
      in_specs=[
          pl.BlockSpec((bm, fp.shape[1]), lambda i: (i, 0)),
          pl.BlockSpec((bm, lp.shape[1]), lambda i: (i, 0)),
          pl.BlockSpec(wf.shape, lambda i: (0, 0)),
          pl.BlockSpec(wl.shape, lambda i: (0, 0)),
      ],
      out_specs=pl.BlockSpec((bm, cout), lambda i: (i, 0)),
      out_shape=jax.ShapeDtypeStruct((np_, cout), jnp.float32),
  )(fp, lp, wf, wl)


def _layer(h, dinv_p, row_p, col_p, w_p, init_w, pool_p, wih, whh, bih, bhh,
           spmm, c):
  score = (h @ pool_p) / jnp.linalg.norm(pool_p)
  score = jnp.where(jnp.arange(_NP) < _N, score, -jnp.inf)
  vals, idx = jax.lax.top_k(score, c)
  x_t = h[idx] * jnp.tanh(vals)[:, None]
  gx = x_t @ wih.T + bih
  gh = init_w @ whh.T + bhh
  r = jax.nn.sigmoid(gx[:, :c] + gh[:, :c])
  z = jax.nn.sigmoid(gx[:, c:2 * c] + gh[:, c:2 * c])
  ng = jnp.tanh(gx[:, 2 * c:] + r * gh[:, 2 * c:])
  w_ev = (1.0 - z) * ng + z * init_w
  xwp = dinv_p[:, None] * _tc_matmul(h, w_ev)
  parts = spmm(row_p, col_p, w_p, xwp)
  return dinv_p[:, None] * (parts[0] + parts[1] + xwp)


def kernel(x, y, edge_index, edge_weight, f_init_w, f_pool, f_Wih, f_Whh,
           f_bih, f_bhh, l_init_w, l_pool, l_Wih, l_Whh, l_bih, l_bhh,
           fusion_W, fusion_b):
  row = edge_index[0]
  col = edge_index[1]
  deg = jax.ops.segment_sum(edge_weight, col, num_segments=_N) + 1.0
  dinv = jax.lax.rsqrt(deg)
  dinv_p = jnp.pad(dinv, (0, _NP - _N))
  row_p = jnp.pad(row, (0, _EP - _E)).reshape(_NW, _EPT)
  col_p = jnp.pad(col, (0, _EP - _E)).reshape(_NW, _EPT)
  w_p = jnp.pad(edge_weight, (0, _EP - _E)).reshape(_NW, _EPT)

  def _tables(c):
    kr = 32 if c == 128 else 128
    shp = (_NW, _EPT // kr, kr)
    return (row_p.reshape(shp), col_p.reshape(shp), w_p.reshape(shp))

  ef = _tables(_CF)
  el = _tables(_CL)

  h = jnp.pad(x, ((0, _NP - _N), (0, 0)))
  hl = jnp.pad(y, ((0, _NP - _N), (0, 0)))
  fi = 0
  for i in range(_NUM_LABEL):
    if fi < _NUM_GCN:
      h = _layer(h, dinv_p, ef[0], ef[1], ef[2], f_init_w[fi], f_pool[fi],
                 f_Wih[fi], f_Whh[fi], f_bih[fi], f_bhh[fi], _spmm_f, _CF)
      fi += 1
    hl = _layer(hl, dinv_p, el[0], el[1], el[2], l_init_w[i], l_pool[i],
                l_Wih[i], l_Whh[i], l_bih[i], l_bhh[i], _spmm_l, _CL)

  wf = fusion_W[:, :_CF].T
  wl = fusion_W[:, _CF:].T
  out = _tc_fusion(h, hl, wf, wl)[:_N] + fusion_b
  return out


# speedup vs baseline: 1.1643x; 1.1372x over previous
import functools

import jax
import jax.numpy as jnp
from jax.experimental import pallas as pl
from jax.experimental.pallas import tpu as pltpu
from jax.experimental.pallas import tpu_sc as plsc

_N = 10000
_E = 320000
_CF = 128
_CL = 64
_NUM_GCN = 2
_NUM_LABEL = 10

_NC = 2
_NS = 16
_NW = _NC * _NS

_NP = 10240
_EP = 327680
_K = 128
_EPT = _EP // _NW
_NCH = _EPT // _K
_RPT = _NP // _NS
_RCH = _RPT // _K


def _make_spmm(C):
  kr = 32 if C == 128 else 80
  nch = _EPT // kr
  rch = _RPT // kr
  src_in_spmem = C == 64
  mesh = plsc.VectorSubcoreMesh(
      core_axis_name="c", subcore_axis_name="s",
      num_cores=_NC, num_subcores=_NS)

  @functools.partial(
      pl.kernel,
      out_type=jax.ShapeDtypeStruct((_NC, _NP, C), jnp.float32),
      mesh=mesh,
      scratch_types=[
          pltpu.VMEM((nch, kr), jnp.int32),
          pltpu.VMEM((nch, kr), jnp.int32),
          pltpu.VMEM((nch, kr), jnp.float32),
          pltpu.VMEM((kr, C), jnp.float32),
          pltpu.VMEM((kr, C), jnp.float32),
          pltpu.VMEM((kr, C), jnp.float32),
          pltpu.VMEM_SHARED((_NP, C), jnp.float32),
          pltpu.VMEM_SHARED((_NP, C) if src_in_spmem else (8, C),
                            jnp.float32),
          pltpu.SemaphoreType.DMA,
          pltpu.SemaphoreType.DMA,
          pltpu.SemaphoreType.DMA,
          pltpu.SemaphoreType.DMA,
          pltpu.SemaphoreType.DMA,
          pltpu.SemaphoreType.DMA,
      ],
      compiler_params=pltpu.CompilerParams(use_tc_tiling_on_sc=False),
  )
  def spmm(row_h, col_h, w_h, xwp_h, out_h, idx_r, idx_c, wbuf, rows0,
           rows1, rows2, acc, xwp_s, semg0, semg1, semg2, sems0, sems1,
           sems2):
    cid = jax.lax.axis_index("c")
    sid = jax.lax.axis_index("s")
    tile = cid * _NS + sid
    rows = (rows0, rows1, rows2)
    semg = (semg0, semg1, semg2)
    sems = (sems0, sems1, sems2)

    pltpu.sync_copy(row_h.at[tile], idx_r)
    pltpu.sync_copy(col_h.at[tile], idx_c)
    pltpu.sync_copy(w_h.at[tile], wbuf)

    if src_in_spmem:
      for b in range(rch):
        r0 = sid * _RPT + b * kr
        pltpu.sync_copy(xwp_h.at[pl.ds(r0, kr)], rows0)
        pltpu.sync_copy(rows0, xwp_s.at[pl.ds(r0, kr)])
      src = xwp_s
    else:
      src = xwp_h

    zero = jnp.zeros((16,), jnp.float32)

    def zrow(i, carry):
      for j in range(C // 16):
        rows0[i, pl.ds(j * 16, 16)] = zero
      return carry

    jax.lax.fori_loop(0, kr, zrow, 0)
    for b in range(rch):
      pltpu.sync_copy(rows0, acc.at[pl.ds(sid * _RPT + b * kr, kr)])
    plsc.subcore_barrier()

    def gather(m, b):
      pltpu.async_copy(src.at[idx_r.at[m]], rows[b], semg[b])

    def wait_gather(m, b):
      pltpu.make_async_copy(src.at[idx_r.at[m]], rows[b], semg[b]).wait()

    def scatter(m, b):
      pltpu.async_copy(rows[b], acc.at[idx_c.at[m]], sems[b], add=True)

    def wait_scatter(m, b):
      pltpu.make_async_copy(rows[b], acc.at[idx_c.at[m]], sems[b]).wait()

    def scale(m, b):
      rb = rows[b]

      def sgroup(g, c2):
        wv = wbuf[m, pl.ds(g * 16, 16)]
        base = g * 16
        for lane in range(16):
          ws = wv[lane]
          for j in range(C // 16):
            sl = pl.ds(j * 16, 16)
            rb[base + lane, sl] = rb[base + lane, sl] * ws
        return c2

      jax.lax.fori_loop(0, kr // 16, sgroup, 0)

    def step(m, b):
      wait_gather(m, b)
      scale(m, b)
      bn = (b + 2) % 3

      @pl.when(m > 0)
      def _():
        wait_scatter(m - 1, bn)

      scatter(m, b)

      @pl.when(m + 2 < nch)
      def _():
        gather(m + 2, bn)

    gather(0, 0)
    gather(1, 1)

    def triple(t, carry):
      m0 = t * 3
      step(m0, 0)
      step(m0 + 1, 1)
      step(m0 + 2, 2)
      return carry

    nt = nch // 3
    jax.lax.fori_loop(0, nt, triple, 0)
    for m in range(nt * 3, nch):
      step(jnp.int32(m), m % 3)
    wait_scatter(jnp.int32(nch - 1), (nch - 1) % 3)
    plsc.subcore_barrier()

    for b in range(rch):
      r0 = sid * _RPT + b * kr
      pltpu.sync_copy(acc.at[pl.ds(r0, kr)], rows0)
      pltpu.sync_copy(rows0, out_h.at[cid, pl.ds(r0, kr)])

  return spmm


_spmm_f = _make_spmm(_CF)
_spmm_l = _make_spmm(_CL)


def _tc_matmul(h, w):
  np_, cin = h.shape
  cout = w.shape[1]
  bm = 1024

  def body(h_ref, w_ref, o_ref):
    o_ref[...] = jnp.dot(h_ref[...], w_ref[...],
                         preferred_element_type=jnp.float32)

  return pl.pallas_call(
      body,
      grid=(np_ // bm,),
      in_specs=[
          pl.BlockSpec((bm, cin), lambda i: (i, 0)),
          pl.BlockSpec((cin, cout), lambda i: (0, 0)),
      ],
      out_specs=pl.BlockSpec((bm, cout), lambda i: (i, 0)),
      out_shape=jax.ShapeDtypeStruct((np_, cout), jnp.float32),
  )(h, w)


def _tc_fusion(fp, lp, wf, wl):
  np_ = fp.shape[0]
  cout = wf.shape[1]
  bm = 1024

  def body(fp_ref, lp_ref, wf_ref, wl_ref, o_ref):
    o_ref[...] = (
        jnp.dot(fp_ref[...], wf_ref[...], preferred_element_type=jnp.float32)
        + jnp.dot(lp_ref[...], wl_ref[...],
                  preferred_element_type=jnp.float32))

  return pl.pallas_call(
      body,
      grid=(np_ // bm,),
      in_specs=[
          pl.BlockSpec((bm, fp.shape[1]), lambda i: (i, 0)),
          pl.BlockSpec((bm, lp.shape[1]), lambda i: (i, 0)),
          pl.BlockSpec(wf.shape, lambda i: (0, 0)),
          pl.BlockSpec(wl.shape, lambda i: (0, 0)),
      ],
      out_specs=pl.BlockSpec((bm, cout), lambda i: (i, 0)),
      out_shape=jax.ShapeDtypeStruct((np_, cout), jnp.float32),
  )(fp, lp, wf, wl)


def _layer(h, dinv_p, row_p, col_p, w_p, init_w, pool_p, wih, whh, bih, bhh,
           spmm, c):
  score = (h @ pool_p) / jnp.linalg.norm(pool_p)
  score = jnp.where(jnp.arange(_NP) < _N, score, -jnp.inf)
  vals, idx = jax.lax.top_k(score, c)
  x_t = h[idx] * jnp.tanh(vals)[:, None]
  gx = x_t @ wih.T + bih
  gh = init_w @ whh.T + bhh
  r = jax.nn.sigmoid(gx[:, :c] + gh[:, :c])
  z = jax.nn.sigmoid(gx[:, c:2 * c] + gh[:, c:2 * c])
  ng = jnp.tanh(gx[:, 2 * c:] + r * gh[:, 2 * c:])
  w_ev = (1.0 - z) * ng + z * init_w
  xwp = dinv_p[:, None] * _tc_matmul(h, w_ev)
  parts = spmm(row_p, col_p, w_p, xwp)
  return dinv_p[:, None] * (parts[0] + parts[1] + xwp)


def kernel(x, y, edge_index, edge_weight, f_init_w, f_pool, f_Wih, f_Whh,
           f_bih, f_bhh, l_init_w, l_pool, l_Wih, l_Whh, l_bih, l_bhh,
           fusion_W, fusion_b):
  row = edge_index[0]
  col = edge_index[1]
  deg = jax.ops.segment_sum(edge_weight, col, num_segments=_N) + 1.0
  dinv = jax.lax.rsqrt(deg)
  dinv_p = jnp.pad(dinv, (0, _NP - _N))
  row_p = jnp.pad(row, (0, _EP - _E)).reshape(_NW, _EPT)
  col_p = jnp.pad(col, (0, _EP - _E)).reshape(_NW, _EPT)
  w_p = jnp.pad(edge_weight, (0, _EP - _E)).reshape(_NW, _EPT)

  def _tables(c):
    kr = 32 if c == 128 else 80
    shp = (_NW, _EPT // kr, kr)
    return (row_p.reshape(shp), col_p.reshape(shp), w_p.reshape(shp))

  ef = _tables(_CF)
  el = _tables(_CL)

  h = jnp.pad(x, ((0, _NP - _N), (0, 0)))
  hl = jnp.pad(y, ((0, _NP - _N), (0, 0)))
  fi = 0
  for i in range(_NUM_LABEL):
    if fi < _NUM_GCN:
      h = _layer(h, dinv_p, ef[0], ef[1], ef[2], f_init_w[fi], f_pool[fi],
                 f_Wih[fi], f_Whh[fi], f_bih[fi], f_bhh[fi], _spmm_f, _CF)
      fi += 1
    hl = _layer(hl, dinv_p, el[0], el[1], el[2], l_init_w[i], l_pool[i],
                l_Wih[i], l_Whh[i], l_bih[i], l_bhh[i], _spmm_l, _CL)

  wf = fusion_W[:, :_CF].T
  wl = fusion_W[:, _CF:].T
  out = _tc_fusion(h, hl, wf, wl)[:_N] + fusion_b
  return out
